# Initial kernel scaffold; baseline (speedup 1.0000x reference)
import jax
import jax.numpy as jnp
from jax.experimental import pallas as pl


def kernel(species, constant):
    raise NotImplementedError("write your pallas kernel here")


# speedup vs baseline: 367.8031x; 367.8031x over previous
import functools

import jax
import jax.numpy as jnp
from jax import lax
from jax.experimental import pallas as pl
from jax.experimental.pallas import tpu as pltpu
from jax.experimental.pallas import tpu_sc as plsc

_N_ATOMS = 4194304
_TABLE_PAD = 128
_CHUNK = 8192


def _build(n_atoms: int):
    info = plsc.get_sparse_core_info()
    nc, ns, nl = info.num_cores, info.num_subcores, info.num_lanes
    nw = nc * ns
    per_w = n_atoms // nw
    n_chunks = per_w // _CHUNK
    assert per_w * nw == n_atoms and n_chunks * _CHUNK == per_w

    mesh = plsc.VectorSubcoreMesh(core_axis_name="c", subcore_axis_name="s")

    @functools.partial(
        pl.kernel,
        mesh=mesh,
        out_type=jax.ShapeDtypeStruct((n_atoms,), jnp.float32),
        compiler_params=pltpu.CompilerParams(needs_layout_passes=False),
        scratch_types=[
            pltpu.VMEM((_TABLE_PAD,), jnp.float32),
            pltpu.VMEM((_CHUNK,), jnp.int32),
            pltpu.VMEM((_CHUNK,), jnp.float32),
        ],
    )
    def gather_kernel(species_hbm, const_hbm, out_hbm, table_v, idx_v, out_v):
        wid = lax.axis_index("s") * nc + lax.axis_index("c")
        base = wid * per_w
        pltpu.sync_copy(const_hbm, table_v)

        def chunk_body(g, carry):
            off = base + g * _CHUNK
            pltpu.sync_copy(species_hbm.at[pl.ds(off, _CHUNK)], idx_v)

            def gather_body(j, c):
                iv = idx_v[pl.ds(j * nl, nl)]
                out_v[pl.ds(j * nl, nl)] = plsc.load_gather(table_v, [iv])
                return c

            lax.fori_loop(0, _CHUNK // nl, gather_body, 0, unroll=8)
            pltpu.sync_copy(out_v, out_hbm.at[pl.ds(off, _CHUNK)])
            return carry

        lax.fori_loop(0, n_chunks, chunk_body, 0)

    return gather_kernel


def kernel(species, constant):
    const_pad = jnp.zeros((_TABLE_PAD,), jnp.float32).at[: constant.shape[0]].set(constant)
    return _build(species.shape[0])(species, const_pad)


# speedup vs baseline: 440.2038x; 1.1968x over previous
import functools

import jax
import jax.numpy as jnp
from jax import lax
from jax.experimental import pallas as pl
from jax.experimental.pallas import tpu as pltpu
from jax.experimental.pallas import tpu_sc as plsc

_N_ATOMS = 4194304
_TABLE_PAD = 128
_CHUNK = 16384
_NBUF = 2


def _build(n_atoms: int):
    info = plsc.get_sparse_core_info()
    nc, ns, nl = info.num_cores, info.num_subcores, info.num_lanes
    nw = nc * ns
    per_w = n_atoms // nw
    n_chunks = per_w // _CHUNK
    assert per_w * nw == n_atoms and n_chunks * _CHUNK == per_w

    mesh = plsc.VectorSubcoreMesh(core_axis_name="c", subcore_axis_name="s")

    @functools.partial(
        pl.kernel,
        mesh=mesh,
        out_type=jax.ShapeDtypeStruct((n_atoms,), jnp.float32),
        compiler_params=pltpu.CompilerParams(needs_layout_passes=False),
        scratch_types=[
            pltpu.VMEM((_TABLE_PAD,), jnp.float32),
            [pltpu.VMEM((_CHUNK,), jnp.int32) for _ in range(_NBUF)],
            [pltpu.VMEM((_CHUNK,), jnp.float32) for _ in range(_NBUF)],
            [pltpu.SemaphoreType.DMA for _ in range(_NBUF)],
            [pltpu.SemaphoreType.DMA for _ in range(_NBUF)],
        ],
    )
    def gather_kernel(species_hbm, const_hbm, out_hbm, table_v, idx_bufs,
                      out_bufs, in_sems, out_sems):
        wid = lax.axis_index("s") * nc + lax.axis_index("c")
        base = wid * per_w
        pltpu.sync_copy(const_hbm, table_v)

        def in_copy(g, b):
            return pltpu.make_async_copy(
                species_hbm.at[pl.ds(base + g * _CHUNK, _CHUNK)],
                idx_bufs[b], in_sems[b])

        def out_copy(g, b):
            return pltpu.make_async_copy(
                out_bufs[b], out_hbm.at[pl.ds(base + g * _CHUNK, _CHUNK)],
                out_sems[b])

        for b in range(min(_NBUF, n_chunks)):
            in_copy(b, b).start()
        for g in range(n_chunks):
            b = g % _NBUF
            in_copy(g, b).wait()
            if g >= _NBUF:
                out_copy(g - _NBUF, b).wait()
            idx_v, out_v = idx_bufs[b], out_bufs[b]

            def gather_body(j, c):
                iv = idx_v[pl.ds(j * nl, nl)]
                out_v[pl.ds(j * nl, nl)] = plsc.load_gather(table_v, [iv])
                return c

            lax.fori_loop(0, _CHUNK // nl, gather_body, 0, unroll=8)
            out_copy(g, b).start()
            if g + _NBUF < n_chunks:
                in_copy(g + _NBUF, b).start()
        for g in range(max(0, n_chunks - _NBUF), n_chunks):
            out_copy(g, g % _NBUF).wait()

    return gather_kernel


def kernel(species, constant):
    const_pad = jnp.zeros((_TABLE_PAD,), jnp.float32).at[: constant.shape[0]].set(constant)
    return _build(species.shape[0])(species, const_pad)


# speedup vs baseline: 1149.0299x; 2.6102x over previous
import functools

import jax
import jax.numpy as jnp
from jax import lax
from jax.experimental import pallas as pl
from jax.experimental.pallas import tpu as pltpu
from jax.experimental.pallas import tpu_sc as plsc

_N_ATOMS = 4194304
_TABLE_PAD = 128
_CHUNK = 16384
_NBUF = 2


def _build(n_atoms: int):
    info = plsc.get_sparse_core_info()
    nc, ns, nl = info.num_cores, info.num_subcores, info.num_lanes
    nw = nc * ns
    per_w = n_atoms // nw
    n_chunks = per_w // _CHUNK
    assert per_w * nw == n_atoms and n_chunks * _CHUNK == per_w

    mesh = plsc.VectorSubcoreMesh(core_axis_name="c", subcore_axis_name="s")

    @functools.partial(
        pl.kernel,
        mesh=mesh,
        out_type=jax.ShapeDtypeStruct((n_atoms,), jnp.float32),
        compiler_params=pltpu.CompilerParams(needs_layout_passes=False),
        scratch_types=[
            pltpu.VMEM((_TABLE_PAD,), jnp.float32),
            [pltpu.VMEM((_CHUNK,), jnp.int32) for _ in range(_NBUF)],
            [pltpu.VMEM((_CHUNK,), jnp.float32) for _ in range(_NBUF)],
            [pltpu.SemaphoreType.DMA for _ in range(_NBUF)],
            [pltpu.SemaphoreType.DMA for _ in range(_NBUF)],
        ],
    )
    def gather_kernel(species_hbm, const_hbm, out_hbm, table_v, idx_bufs,
                      out_bufs, in_sems, out_sems):
        wid = lax.axis_index("s") * nc + lax.axis_index("c")
        base = wid * per_w
        pltpu.sync_copy(const_hbm, table_v)

        def in_copy(g, b):
            return pltpu.make_async_copy(
                species_hbm.at[pl.ds(base + g * _CHUNK, _CHUNK)],
                idx_bufs[b], in_sems[b])

        def out_copy(g, b):
            return pltpu.make_async_copy(
                out_bufs[b], out_hbm.at[pl.ds(base + g * _CHUNK, _CHUNK)],
                out_sems[b])

        for b in range(min(_NBUF, n_chunks)):
            in_copy(b, b).start()
        for g in range(n_chunks):
            b = g % _NBUF
            in_copy(g, b).wait()
            if g >= _NBUF:
                out_copy(g - _NBUF, b).wait()
            idx_v, out_v = idx_bufs[b], out_bufs[b]

            @plsc.parallel_loop(0, _CHUNK, step=nl, unroll=8)
            def gather_body(i):
                iv = idx_v[pl.ds(i, nl)]
                out_v[pl.ds(i, nl)] = plsc.load_gather(table_v, [iv])
            out_copy(g, b).start()
            if g + _NBUF < n_chunks:
                in_copy(g + _NBUF, b).start()
        for g in range(max(0, n_chunks - _NBUF), n_chunks):
            out_copy(g, g % _NBUF).wait()

    return gather_kernel


def kernel(species, constant):
    const_pad = jnp.zeros((_TABLE_PAD,), jnp.float32).at[: constant.shape[0]].set(constant)
    return _build(species.shape[0])(species, const_pad)
